# Initial kernel scaffold; baseline (speedup 1.0000x reference)
import jax
import jax.numpy as jnp
from jax.experimental import pallas as pl


def kernel(l, u, in_l, in_u):
    raise NotImplementedError("write your pallas kernel here")


# speedup vs baseline: 1.8229x; 1.8229x over previous
import functools

import jax
import jax.numpy as jnp
from jax.experimental import pallas as pl


def _body(cr_ref, l_ref, u_ref, lout_ref, uout_ref, plb_ref, pub_ref, *, D):
    lb = l_ref[...]
    ub = u_ref[...]
    cp = cr_ref[0:1, :]
    rp = cr_ref[1:2, :]

    s_l = jnp.sum(lb * cp, axis=1)
    t_l = jnp.sum(jnp.abs(lb) * rp, axis=1)
    s_u = jnp.sum(ub * cp, axis=1)
    t_u = jnp.sum(jnp.abs(ub) * rp, axis=1)

    conc_lb = s_l - t_l
    max_lb = s_l + t_l
    conc_ub = s_u + t_u
    min_ub = s_u - t_u

    inactive = conc_ub <= 0.0
    unstable = (conc_lb < 0.0) & (conc_ub > 0.0)
    mostly_inactive = unstable & (
        (jnp.abs(conc_lb) > jnp.abs(conc_ub)) | (max_lb <= 0.0))
    mostly_active = unstable & (jnp.abs(conc_lb) <= jnp.abs(conc_ub))

    denom_l = jnp.where(unstable, max_lb - conc_lb, 1.0)
    a_l = jnp.where(max_lb < 0.0, 0.0, max_lb / denom_l)
    scale_l = jnp.where(inactive | mostly_inactive, 0.0, 1.0)
    scale_l = jnp.where(mostly_active, a_l, scale_l)

    zero_crossing = unstable & (min_ub <= 0.0)
    denom_u = jnp.where(zero_crossing, conc_ub - min_ub, 1.0)
    a_u = conc_ub / denom_u
    scale_u = jnp.where(inactive, 0.0, 1.0)
    scale_u = jnp.where(zero_crossing, a_u, scale_u)
    bias_adj = jnp.where(zero_crossing, a_u * min_ub, 0.0)

    lout_ref[...] = scale_l[:, None] * lb
    u_scaled = scale_u[:, None] * ub
    col = jax.lax.broadcasted_iota(jnp.int32, u_scaled.shape, 1)
    uout_ref[...] = jnp.where(col == D - 1,
                              u_scaled - bias_adj[:, None], u_scaled)
    plb_ref[...] = (scale_l * conc_lb)[:, None]
    pub_ref[...] = (scale_u * conc_ub - bias_adj)[:, None]


def kernel(l, u, in_l, in_u):
    B, N, D = l.shape
    M = B * N
    c = (in_l + in_u) * 0.5
    r = (in_u - in_l) * 0.5
    one = jnp.ones((1,), l.dtype)
    zero = jnp.zeros((1,), l.dtype)
    cr = jnp.stack([jnp.concatenate([c, one]), jnp.concatenate([r, zero])])

    l2 = l.reshape(M, D)
    u2 = u.reshape(M, D)

    BLK = 512
    grid = (M // BLK,)
    out = pl.pallas_call(
        functools.partial(_body, D=D),
        grid=grid,
        in_specs=[
            pl.BlockSpec((2, D), lambda i: (0, 0)),
            pl.BlockSpec((BLK, D), lambda i: (i, 0)),
            pl.BlockSpec((BLK, D), lambda i: (i, 0)),
        ],
        out_specs=[
            pl.BlockSpec((BLK, D), lambda i: (i, 0)),
            pl.BlockSpec((BLK, D), lambda i: (i, 0)),
            pl.BlockSpec((BLK, 1), lambda i: (i, 0)),
            pl.BlockSpec((BLK, 1), lambda i: (i, 0)),
        ],
        out_shape=[
            jax.ShapeDtypeStruct((M, D), l.dtype),
            jax.ShapeDtypeStruct((M, D), l.dtype),
            jax.ShapeDtypeStruct((M, 1), l.dtype),
            jax.ShapeDtypeStruct((M, 1), l.dtype),
        ],
    )(cr, l2, u2)
    l_new, u_new, post_lb, post_ub = out
    return (l_new.reshape(B, N, D), u_new.reshape(B, N, D),
            post_lb.reshape(B, N), post_ub.reshape(B, N))
